# baseline (speedup 1.0000x reference)
import jax
import jax.numpy as jnp
from jax.experimental import pallas as pl

TEMPERATURE = 8.0
BI = 256


def _deep_attn_kernel(qt_ref, kt_ref, v_ref, mf_ref,
                      W1qT_ref, W1kT_ref, b1c_ref, W2Ta_ref,
                      w3p_ref, b3s_ref,
                      out_ref, attn_ref):
    qt = qt_ref[0]
    kt = kt_ref[0]
    v = v_ref[0]
    mf = mf_ref[0]
    S = kt.shape[1]

    aT = (jnp.dot(W1qT_ref[...], qt, preferred_element_type=jnp.float32)
          + b1c_ref[...]).astype(jnp.bfloat16)
    cT = jnp.dot(W1kT_ref[...], kt,
                 preferred_element_type=jnp.float32).astype(jnp.bfloat16)

    h1 = jnp.maximum(jnp.repeat(aT, S, axis=1) + jnp.tile(cT, (1, BI)),
                     jnp.bfloat16(0))
    h1a = jnp.concatenate(
        [h1, jnp.ones((1, BI * S), jnp.bfloat16),
         jnp.zeros((15, BI * S), jnp.bfloat16)], axis=0)
    W2a = W2Ta_ref[...].astype(jnp.bfloat16)
    h2 = jnp.maximum(
        jnp.dot(W2a, h1a, preferred_element_type=jnp.float32), 0.0)
    head = jnp.dot(w3p_ref[...], h2,
                   preferred_element_type=jnp.float32)
    scores = head[:1, :].reshape(BI, S)

    logits = scores + b3s_ref[0, 0] + (mf - 1.0) * 1e16
    mx = jnp.max(logits, axis=1, keepdims=True)
    e = jnp.exp(logits - mx)
    attn = e / jnp.sum(e, axis=1, keepdims=True)

    attn_ref[0] = attn
    out_ref[0] = jnp.dot(attn, v, preferred_element_type=jnp.float32)


def kernel(q, k, v, mask, W1, b1, W2, b2, W3, b3):
    B, S, T = q.shape
    qt = jnp.swapaxes(q, 1, 2)
    kt = jnp.swapaxes(k, 1, 2)
    mf = mask.astype(jnp.float32)
    W1qT = W1[:T].T
    W1kT = W1[T:].T
    b1c = b1.reshape(-1, 1)
    W2Ta = jnp.pad(jnp.concatenate([W2.T, b2.reshape(-1, 1)], axis=1),
                   ((0, 0), (0, 15)))
    w3p = jnp.pad((W3 * (1.0 / TEMPERATURE)).T, ((0, 7), (0, 0)))
    b3s = (b3 * (1.0 / TEMPERATURE)).reshape(1, 1)

    grid = (B, S // BI)
    out_shape = (
        jax.ShapeDtypeStruct((B, S, T), jnp.float32),
        jax.ShapeDtypeStruct((B, S, S), jnp.float32),
    )
    full = lambda shape: pl.BlockSpec(shape, lambda b, i: (0,) * len(shape))
    out, attn = pl.pallas_call(
        _deep_attn_kernel,
        grid=grid,
        in_specs=[
            pl.BlockSpec((1, T, BI), lambda b, i: (b, 0, i)),
            pl.BlockSpec((1, T, S), lambda b, i: (b, 0, 0)),
            pl.BlockSpec((1, S, T), lambda b, i: (b, 0, 0)),
            pl.BlockSpec((1, BI, S), lambda b, i: (b, i, 0)),
            full((80, T)),
            full((80, T)),
            full((80, 1)),
            full((60, 96)),
            full((8, 60)),
            full((1, 1)),
        ],
        out_specs=[
            pl.BlockSpec((1, BI, T), lambda b, i: (b, i, 0)),
            pl.BlockSpec((1, BI, S), lambda b, i: (b, i, 0)),
        ],
        out_shape=out_shape,
    )(qt, kt, v, mf, W1qT, W1kT, b1c, W2Ta, w3p, b3s)
    return (out, attn)


# speedup vs baseline: 1.0936x; 1.0936x over previous
import jax
import jax.numpy as jnp
from jax.experimental import pallas as pl

TEMPERATURE = 8.0
BI = 256


def _deep_attn_kernel(qt_ref, kt_ref, v_ref, mf_ref,
                      W1qT_ref, W1kT_ref, b1c_ref, W2T_ref, b2c_ref,
                      w3p_ref, b3s_ref,
                      out_ref, attn_ref):
    qt = qt_ref[0]
    kt = kt_ref[0]
    v = v_ref[0]
    mf = mf_ref[0]
    S = kt.shape[1]

    aT = (jnp.dot(W1qT_ref[...], qt, preferred_element_type=jnp.float32)
          + b1c_ref[...]).astype(jnp.bfloat16)
    cT = jnp.dot(W1kT_ref[...], kt,
                 preferred_element_type=jnp.float32).astype(jnp.bfloat16)

    h1 = jnp.maximum(jnp.repeat(aT, S, axis=1) + jnp.tile(cT, (1, BI)),
                     jnp.bfloat16(0))
    W2b = W2T_ref[...].astype(jnp.bfloat16)
    h2 = jnp.maximum(
        jnp.dot(W2b, h1, preferred_element_type=jnp.float32) + b2c_ref[...],
        0.0)
    head = jnp.dot(w3p_ref[...], h2,
                   preferred_element_type=jnp.float32)
    scores = head[:1, :].reshape(BI, S)

    logits = scores + b3s_ref[0, 0] + (mf - 1.0) * 1e16
    mx = jnp.max(logits, axis=1, keepdims=True)
    e = jnp.exp(logits - mx)
    attn = e / jnp.sum(e, axis=1, keepdims=True)

    attn_ref[0] = attn
    out_ref[0] = jnp.dot(attn, v, preferred_element_type=jnp.float32)


def kernel(q, k, v, mask, W1, b1, W2, b2, W3, b3):
    B, S, T = q.shape
    qt = jnp.swapaxes(q, 1, 2)
    kt = jnp.swapaxes(k, 1, 2)
    mf = mask.astype(jnp.float32)
    W1qT = W1[:T].T
    W1kT = W1[T:].T
    b1c = b1.reshape(-1, 1)
    W2T = W2.T
    b2c = b2.reshape(-1, 1)
    w3p = jnp.pad((W3 * (1.0 / TEMPERATURE)).T, ((0, 7), (0, 0)))
    b3s = (b3 * (1.0 / TEMPERATURE)).reshape(1, 1)

    grid = (B, S // BI)
    out_shape = (
        jax.ShapeDtypeStruct((B, S, T), jnp.float32),
        jax.ShapeDtypeStruct((B, S, S), jnp.float32),
    )
    full = lambda shape: pl.BlockSpec(shape, lambda b, i: (0,) * len(shape))
    out, attn = pl.pallas_call(
        _deep_attn_kernel,
        grid=grid,
        in_specs=[
            pl.BlockSpec((1, T, BI), lambda b, i: (b, 0, i)),
            pl.BlockSpec((1, T, S), lambda b, i: (b, 0, 0)),
            pl.BlockSpec((1, S, T), lambda b, i: (b, 0, 0)),
            pl.BlockSpec((1, BI, S), lambda b, i: (b, i, 0)),
            full((80, T)),
            full((80, T)),
            full((80, 1)),
            full((60, 80)),
            full((60, 1)),
            full((8, 60)),
            full((1, 1)),
        ],
        out_specs=[
            pl.BlockSpec((1, BI, T), lambda b, i: (b, i, 0)),
            pl.BlockSpec((1, BI, S), lambda b, i: (b, i, 0)),
        ],
        out_shape=out_shape,
    )(qt, kt, v, mf, W1qT, W1kT, b1c, W2T, b2c, w3p, b3s)
    return (out, attn)
